# Initial kernel scaffold; baseline (speedup 1.0000x reference)
import jax
import jax.numpy as jnp
from jax.experimental import pallas as pl


def kernel(n_feats, edge_index, W1, b1, W2, b2):
    raise NotImplementedError("write your pallas kernel here")


# speedup vs baseline: 7.4335x; 7.4335x over previous
import functools

import jax
import jax.numpy as jnp
from jax import lax
from jax.experimental import pallas as pl
from jax.experimental.pallas import tpu as pltpu
from jax.experimental.pallas import tpu_sc as plsc

N = 10000
E = 320000
F_IN = 128
H = 128
C = 64

NPAD = 10240
NW = 32
EPW = E // NW
CHW = 128
CH = -(-EPW // CHW)
EPW_PAD = CH * CHW
DCH = -(-(2 * EPW) // CHW)
DPW_PAD = DCH * CHW

_mesh = plsc.VectorSubcoreMesh(core_axis_name="c", subcore_axis_name="s")


@functools.partial(
    pl.kernel,
    mesh=_mesh,
    out_type=jax.ShapeDtypeStruct((2, 2 * NPAD), jnp.float32),
    scratch_types=[
        pltpu.VMEM((DCH, CHW), jnp.int32),
        pltpu.VMEM((CHW,), jnp.float32),
        pltpu.VMEM_SHARED((2 * NPAD,), jnp.float32),
    ],
)
def _sc_degrees(idx_hbm, ones_hbm, zeros_hbm, out_hbm, idx_v, ones_v, acc_sh):
    c = lax.axis_index("c")
    s = lax.axis_index("s")
    wid = s * 2 + c
    sl = (2 * NPAD) // 16
    pltpu.sync_copy(zeros_hbm.at[pl.ds(s * sl, sl)], acc_sh.at[pl.ds(s * sl, sl)])
    pltpu.sync_copy(ones_hbm, ones_v)
    pltpu.sync_copy(idx_hbm.at[wid], idx_v)
    plsc.subcore_barrier()

    def body(j, carry):
        pltpu.sync_copy(ones_v, acc_sh.at[idx_v.at[j]], add=True)
        return carry

    lax.fori_loop(0, DCH, body, 0)
    plsc.subcore_barrier()
    pltpu.sync_copy(acc_sh.at[pl.ds(s * sl, sl)], out_hbm.at[c, pl.ds(s * sl, sl)])


def _make_sc_agg(F):
    @functools.partial(
        pl.kernel,
        mesh=_mesh,
        out_type=jax.ShapeDtypeStruct((2, NPAD, F), jnp.float32),
        scratch_types=[
            pltpu.VMEM((CH, CHW), jnp.int32),
            pltpu.VMEM((CH, CHW), jnp.int32),
            pltpu.VMEM((CHW, F), jnp.float32),
            pltpu.VMEM_SHARED((NPAD, F), jnp.float32),
            pltpu.SemaphoreType.DMA,
        ],
    )
    def _sc_agg(table_hbm, sidx_hbm, didx_hbm, zeros_hbm, out_hbm,
                sidx_v, didx_v, rows_v, acc_sh, sem):
        c = lax.axis_index("c")
        s = lax.axis_index("s")
        wid = s * 2 + c
        rs = NPAD // 16
        pltpu.sync_copy(zeros_hbm.at[pl.ds(s * rs, rs)], acc_sh.at[pl.ds(s * rs, rs)])
        pltpu.sync_copy(sidx_hbm.at[wid], sidx_v)
        pltpu.sync_copy(didx_hbm.at[wid], didx_v)
        plsc.subcore_barrier()

        def body(j, carry):
            pltpu.async_copy(table_hbm.at[sidx_v.at[j]], rows_v, sem).wait()
            pltpu.sync_copy(rows_v, acc_sh.at[didx_v.at[j]], add=True)
            return carry

        lax.fori_loop(0, CH, body, 0)
        plsc.subcore_barrier()
        pltpu.sync_copy(acc_sh.at[pl.ds(s * rs, rs)],
                        out_hbm.at[c, pl.ds(s * rs, rs)])

    return _sc_agg


_sc_agg_h = _make_sc_agg(H)


def _norm_cols(deg_ref):
    d = deg_ref[...]
    deg_out = jnp.maximum(d[:, 0:1] + d[:, 2:3], 1.0)
    deg_in = jnp.maximum(d[:, 1:2] + d[:, 3:4], 1.0)
    return lax.rsqrt(deg_out)[0:N], lax.rsqrt(deg_in)[0:N]


def _tc_prep_body(x_ref, w_ref, deg_ref, o_ref):
    norm_src, _ = _norm_cols(deg_ref)
    x = x_ref[...] * norm_src
    o_ref[...] = jnp.dot(x, w_ref[...], preferred_element_type=jnp.float32)


def _tc_mid_body(aggp_ref, deg_ref, b1_ref, o_ref):
    norm_src, norm_dst = _norm_cols(deg_ref)
    agg = aggp_ref[0, 0:N, :] + aggp_ref[1, 0:N, :]
    h = jnp.maximum(agg * norm_dst + b1_ref[...], 0.0)
    o_ref[...] = h * norm_src


def _tc_fin_body(aggp_ref, deg_ref, w2_ref, b2_ref, o_ref):
    _, norm_dst = _norm_cols(deg_ref)
    agg = aggp_ref[0, 0:N, :] + aggp_ref[1, 0:N, :]
    o_ref[...] = (jnp.dot(agg, w2_ref[...], preferred_element_type=jnp.float32)
                  * norm_dst + b2_ref[...])


def kernel(n_feats, edge_index, W1, b1, W2, b2):
    src = edge_index[0].astype(jnp.int32)
    dst = edge_index[1].astype(jnp.int32)

    srcw = src.reshape(NW, EPW)
    dstw = dst.reshape(NW, EPW)
    padn = EPW_PAD - EPW
    pad_src = (jnp.arange(padn, dtype=jnp.int32) * 89) % N
    pad_dst = N + (jnp.arange(padn, dtype=jnp.int32) % (NPAD - N))
    src_p = jnp.concatenate(
        [srcw, jnp.broadcast_to(pad_src, (NW, padn))], axis=1
    ).reshape(NW, CH, CHW)
    dst_p = jnp.concatenate(
        [dstw, jnp.broadcast_to(pad_dst, (NW, padn))], axis=1
    ).reshape(NW, CH, CHW)

    dpadn = DPW_PAD - 2 * EPW
    pad_deg = 2 * N + ((jnp.arange(dpadn, dtype=jnp.int32) * 3) % (2 * (NPAD - N)))
    deg_idx = jnp.concatenate(
        [2 * srcw, 2 * dstw + 1, jnp.broadcast_to(pad_deg, (NW, dpadn))], axis=1
    ).reshape(NW, DCH, CHW)

    ones_chunk = jnp.ones((CHW,), jnp.float32)
    zeros_deg = jnp.zeros((2 * NPAD,), jnp.float32)
    zeros_h = jnp.zeros((NPAD, H), jnp.float32)

    deg2 = _sc_degrees(deg_idx, ones_chunk, zeros_deg)
    deg_cols = deg2.reshape(2, NPAD, 2).transpose(1, 0, 2).reshape(NPAD, 4)

    xw = pl.pallas_call(
        _tc_prep_body,
        out_shape=jax.ShapeDtypeStruct((N, H), jnp.float32),
    )(n_feats, W1, deg_cols)

    agg1p = _sc_agg_h(xw, src_p, dst_p, zeros_h)

    hs = pl.pallas_call(
        _tc_mid_body,
        out_shape=jax.ShapeDtypeStruct((N, H), jnp.float32),
    )(agg1p, deg_cols, b1.reshape(1, H))

    agg2p = _sc_agg_h(hs, src_p, dst_p, zeros_h)

    out = pl.pallas_call(
        _tc_fin_body,
        out_shape=jax.ShapeDtypeStruct((N, C), jnp.float32),
    )(agg2p, deg_cols, W2, b2.reshape(1, C))

    return out


# speedup vs baseline: 9.0159x; 1.2129x over previous
import functools

import jax
import jax.numpy as jnp
from jax import lax
from jax.experimental import pallas as pl
from jax.experimental.pallas import tpu as pltpu
from jax.experimental.pallas import tpu_sc as plsc

N = 10000
E = 320000
F_IN = 128
H = 128
C = 64

NPAD = 10240
NW = 32
EPW = E // NW
CHW = 128
CH = 80
EPW_PAD = CH * CHW
DCH = -(-(2 * EPW) // CHW)
DPW_PAD = DCH * CHW

_mesh = plsc.VectorSubcoreMesh(core_axis_name="c", subcore_axis_name="s")


@functools.partial(
    pl.kernel,
    mesh=_mesh,
    out_type=jax.ShapeDtypeStruct((2, 2 * NPAD), jnp.float32),
    scratch_types=[
        pltpu.VMEM((DCH, CHW), jnp.int32),
        pltpu.VMEM((CHW,), jnp.float32),
        pltpu.VMEM_SHARED((2 * NPAD,), jnp.float32),
    ],
)
def _sc_degrees(idx_hbm, ones_hbm, zeros_hbm, out_hbm, idx_v, ones_v, acc_sh):
    c = lax.axis_index("c")
    s = lax.axis_index("s")
    wid = s * 2 + c
    sl = (2 * NPAD) // 16
    pltpu.sync_copy(zeros_hbm.at[pl.ds(s * sl, sl)], acc_sh.at[pl.ds(s * sl, sl)])
    pltpu.sync_copy(ones_hbm, ones_v)
    pltpu.sync_copy(idx_hbm.at[wid], idx_v)
    plsc.subcore_barrier()

    def body(j, carry):
        pltpu.sync_copy(ones_v, acc_sh.at[idx_v.at[j]], add=True)
        return carry

    lax.fori_loop(0, DCH, body, 0)
    plsc.subcore_barrier()
    pltpu.sync_copy(acc_sh.at[pl.ds(s * sl, sl)], out_hbm.at[c, pl.ds(s * sl, sl)])


def _make_sc_agg(F):
    @functools.partial(
        pl.kernel,
        mesh=_mesh,
        out_type=jax.ShapeDtypeStruct((2, NPAD, F), jnp.float32),
        scratch_types=[
            pltpu.VMEM((CH // 2, CHW), jnp.int32),
            pltpu.VMEM((CH // 2, CHW), jnp.int32),
            pltpu.VMEM((CHW, F), jnp.float32),
            pltpu.VMEM((CHW, F), jnp.float32),
            pltpu.VMEM_SHARED((NPAD, F), jnp.float32),
            pltpu.SemaphoreType.DMA,
        ],
    )
    def _sc_agg(table_hbm, sidx_hbm, didx_hbm, zeros_hbm, out_hbm,
                sidx_v, didx_v, rows0_v, rows1_v, acc_sh, sem):
        c = lax.axis_index("c")
        s = lax.axis_index("s")
        wid = s * 2 + c
        rs = NPAD // 16
        cpp = CH // 2
        pltpu.sync_copy(zeros_hbm.at[pl.ds(s * rs, rs)], acc_sh.at[pl.ds(s * rs, rs)])
        plsc.subcore_barrier()

        for p in range(2):
            pltpu.sync_copy(sidx_hbm.at[wid, pl.ds(p * cpp, cpp)], sidx_v)
            pltpu.sync_copy(didx_hbm.at[wid, pl.ds(p * cpp, cpp)], didx_v)

            pltpu.async_copy(table_hbm.at[sidx_v.at[0]], rows0_v, sem).wait()

            def body(j2, carry):
                j = j2 * 2
                pltpu.async_copy(table_hbm.at[sidx_v.at[j + 1]], rows1_v, sem)
                pltpu.sync_copy(rows0_v, acc_sh.at[didx_v.at[j]], add=True)
                pltpu.make_async_copy(table_hbm.at[sidx_v.at[j + 1]], rows1_v,
                                      sem).wait()

                @pl.when(j2 < cpp // 2 - 1)
                def _():
                    pltpu.async_copy(table_hbm.at[sidx_v.at[j + 2]], rows0_v,
                                     sem)

                pltpu.sync_copy(rows1_v, acc_sh.at[didx_v.at[j + 1]], add=True)

                @pl.when(j2 < cpp // 2 - 1)
                def _():
                    pltpu.make_async_copy(table_hbm.at[sidx_v.at[j + 2]],
                                          rows0_v, sem).wait()
                return carry

            lax.fori_loop(0, cpp // 2, body, 0)
        plsc.subcore_barrier()
        pltpu.sync_copy(acc_sh.at[pl.ds(s * rs, rs)],
                        out_hbm.at[c, pl.ds(s * rs, rs)])

    return _sc_agg


_sc_agg_h = _make_sc_agg(H)


def _norm_cols(deg_ref):
    d = deg_ref[...]
    deg_out = jnp.maximum(d[:, 0:1] + d[:, 2:3], 1.0)
    deg_in = jnp.maximum(d[:, 1:2] + d[:, 3:4], 1.0)
    return lax.rsqrt(deg_out)[0:N], lax.rsqrt(deg_in)[0:N]


def _tc_prep_body(x_ref, w_ref, deg_ref, o_ref):
    norm_src, _ = _norm_cols(deg_ref)
    x = x_ref[...] * norm_src
    o_ref[...] = jnp.dot(x, w_ref[...], preferred_element_type=jnp.float32)


def _tc_mid_body(aggp_ref, deg_ref, b1_ref, o_ref):
    norm_src, norm_dst = _norm_cols(deg_ref)
    agg = aggp_ref[0, 0:N, :] + aggp_ref[1, 0:N, :]
    h = jnp.maximum(agg * norm_dst + b1_ref[...], 0.0)
    o_ref[...] = h * norm_src


def _tc_fin_body(aggp_ref, deg_ref, w2_ref, b2_ref, o_ref):
    _, norm_dst = _norm_cols(deg_ref)
    agg = aggp_ref[0, 0:N, :] + aggp_ref[1, 0:N, :]
    o_ref[...] = (jnp.dot(agg, w2_ref[...], preferred_element_type=jnp.float32)
                  * norm_dst + b2_ref[...])


def kernel(n_feats, edge_index, W1, b1, W2, b2):
    src = edge_index[0].astype(jnp.int32)
    dst = edge_index[1].astype(jnp.int32)

    srcw = src.reshape(NW, EPW)
    dstw = dst.reshape(NW, EPW)
    padn = EPW_PAD - EPW
    pad_src = (jnp.arange(padn, dtype=jnp.int32) * 89) % N
    pad_dst = N + (jnp.arange(padn, dtype=jnp.int32) % (NPAD - N))
    src_p = jnp.concatenate(
        [srcw, jnp.broadcast_to(pad_src, (NW, padn))], axis=1
    ).reshape(NW, CH, CHW)
    dst_p = jnp.concatenate(
        [dstw, jnp.broadcast_to(pad_dst, (NW, padn))], axis=1
    ).reshape(NW, CH, CHW)

    dpadn = DPW_PAD - 2 * EPW
    pad_deg = 2 * N + ((jnp.arange(dpadn, dtype=jnp.int32) * 3) % (2 * (NPAD - N)))
    deg_idx = jnp.concatenate(
        [2 * srcw, 2 * dstw + 1, jnp.broadcast_to(pad_deg, (NW, dpadn))], axis=1
    ).reshape(NW, DCH, CHW)

    ones_chunk = jnp.ones((CHW,), jnp.float32)
    zeros_deg = jnp.zeros((2 * NPAD,), jnp.float32)
    zeros_h = jnp.zeros((NPAD, H), jnp.float32)

    deg2 = _sc_degrees(deg_idx, ones_chunk, zeros_deg)
    deg_cols = deg2.reshape(2, NPAD, 2).transpose(1, 0, 2).reshape(NPAD, 4)

    xw = pl.pallas_call(
        _tc_prep_body,
        out_shape=jax.ShapeDtypeStruct((N, H), jnp.float32),
    )(n_feats, W1, deg_cols)

    agg1p = _sc_agg_h(xw, src_p, dst_p, zeros_h)

    hs = pl.pallas_call(
        _tc_mid_body,
        out_shape=jax.ShapeDtypeStruct((N, H), jnp.float32),
    )(agg1p, deg_cols, b1.reshape(1, H))

    agg2p = _sc_agg_h(hs, src_p, dst_p, zeros_h)

    out = pl.pallas_call(
        _tc_fin_body,
        out_shape=jax.ShapeDtypeStruct((N, C), jnp.float32),
    )(agg2p, deg_cols, W2, b2.reshape(1, C))

    return out
